# Initial kernel scaffold; baseline (speedup 1.0000x reference)
import jax
import jax.numpy as jnp
from jax.experimental import pallas as pl


def kernel(x, token_table, pos_table):
    raise NotImplementedError("write your pallas kernel here")


# speedup vs baseline: 4.6290x; 4.6290x over previous
import functools

import jax
import jax.numpy as jnp
from jax import lax
from jax.experimental import pallas as pl
from jax.experimental.pallas import tpu as pltpu
from jax.experimental.pallas import tpu_sc as plsc

BATCH = 4096
MAXLEN = 200
EMBED = 32

NC = 2
NS = 16
NW = NC * NS

ROWS_PER_W = BATCH // NW
CHUNK_ROWS = 4
CHUNK = CHUNK_ROWS * MAXLEN
NCHUNK = ROWS_PER_W // CHUNK_ROWS
GATHER_W = 100
NGATHER = CHUNK // GATHER_W


def _embed_kernel(x_hbm, tok_hbm, pos_hbm, out_hbm, idx_v, rows_v, pos_v, sem):
    wid = lax.axis_index("c") * NS + lax.axis_index("s")

    pltpu.sync_copy(pos_hbm, pos_v)

    def chunk_body(g, carry):
        c = wid * NCHUNK + g
        pltpu.sync_copy(x_hbm.at[c], idx_v)

        copies = []
        for j in range(NGATHER):
            copies.append(
                pltpu.async_copy(
                    tok_hbm.at[idx_v.at[j]],
                    rows_v.at[pl.ds(j * GATHER_W, GATHER_W)],
                    sem,
                )
            )
        for cp in copies:
            cp.wait()

        def add_body(t, carry2):
            p0 = pos_v[t, pl.ds(0, 16)]
            p1 = pos_v[t, pl.ds(16, 16)]
            for r in range(CHUNK_ROWS):
                row = r * MAXLEN + t
                rows_v[row, pl.ds(0, 16)] = rows_v[row, pl.ds(0, 16)] + p0
                rows_v[row, pl.ds(16, 16)] = rows_v[row, pl.ds(16, 16)] + p1
            return carry2

        lax.fori_loop(0, MAXLEN, add_body, 0)

        pltpu.sync_copy(rows_v, out_hbm.at[pl.ds(c * CHUNK, CHUNK)])
        return carry

    lax.fori_loop(0, NCHUNK, chunk_body, 0)


def kernel(x, token_table, pos_table):
    x3 = x.astype(jnp.int32).reshape(NW * NCHUNK, NGATHER, GATHER_W)
    mesh = plsc.VectorSubcoreMesh(core_axis_name="c", subcore_axis_name="s")
    run = functools.partial(
        pl.kernel,
        mesh=mesh,
        compiler_params=pltpu.CompilerParams(use_tc_tiling_on_sc=False),
        out_type=jax.ShapeDtypeStruct((BATCH * MAXLEN, EMBED), jnp.float32),
        scratch_types=[
            pltpu.VMEM((NGATHER, GATHER_W), jnp.int32),
            pltpu.VMEM((CHUNK, EMBED), jnp.float32),
            pltpu.VMEM((MAXLEN, EMBED), jnp.float32),
            pltpu.SemaphoreType.DMA,
        ],
    )(_embed_kernel)
    out = run(x3, token_table, pos_table)
    return out.reshape(BATCH, MAXLEN, EMBED)
